# baseline (speedup 1.0000x reference)
import functools

import jax
import jax.numpy as jnp
from jax import lax
from jax.experimental import pallas as pl
from jax.experimental.pallas import tpu as pltpu
from jax.experimental.pallas import tpu_sc as plsc


def _copy_kernel(te_ref, out_ref):
    out_ref[...] = te_ref[...]


def _make_sc_pos(batch, seq, d, dtype):
    info = plsc.get_sparse_core_info()
    nc, ns = info.num_cores, info.num_subcores
    nw = nc * ns
    chunks_per_b = nw // batch
    rows = seq // chunks_per_b
    mesh = plsc.VectorSubcoreMesh(core_axis_name="c", subcore_axis_name="s")

    @functools.partial(
        pl.kernel,
        mesh=mesh,
        out_type=jax.ShapeDtypeStruct((batch * seq, d), dtype),
    )
    def sc_pos(w_hbm, out_hbm):
        wid = lax.axis_index("s") * nc + lax.axis_index("c")
        b = wid // chunks_per_b
        s0 = (wid % chunks_per_b) * rows
        pltpu.sync_copy(
            w_hbm.at[pl.ds(s0, rows)],
            out_hbm.at[pl.ds(b * seq + s0, rows)],
        )

    return sc_pos


def kernel(tokens, token_embed, W_pos):
    batch, seq, d = token_embed.shape
    pos_flat = _make_sc_pos(batch, seq, d, W_pos.dtype)(W_pos)
    pos_embed = pos_flat.reshape(batch, seq, d)

    block_s = 512
    te_out = pl.pallas_call(
        _copy_kernel,
        grid=(seq // block_s,),
        in_specs=[pl.BlockSpec((batch, block_s, d), lambda j: (0, j, 0))],
        out_specs=pl.BlockSpec((batch, block_s, d), lambda j: (0, j, 0)),
        out_shape=jax.ShapeDtypeStruct((batch, seq, d), token_embed.dtype),
    )(token_embed)
    return (pos_embed, te_out)


# speedup vs baseline: 24.0542x; 24.0542x over previous
import functools

import jax
import jax.numpy as jnp
from jax import lax
from jax.experimental import pallas as pl
from jax.experimental.pallas import tpu as pltpu
from jax.experimental.pallas import tpu_sc as plsc


def _copy_kernel(te_ref, out_ref):
    out_ref[...] = te_ref[...]


def _make_sc_pos(batch, seq, d, dtype):
    info = plsc.get_sparse_core_info()
    nc, ns = info.num_cores, info.num_subcores
    nw = nc * ns
    rows = seq // nw
    chunk = 64
    n_chunks = rows // chunk
    mesh = plsc.VectorSubcoreMesh(core_axis_name="c", subcore_axis_name="s")

    @functools.partial(
        pl.kernel,
        mesh=mesh,
        out_type=jax.ShapeDtypeStruct((batch * seq, d), dtype),
        scratch_types=[pltpu.VMEM((chunk, d), dtype)],
    )
    def sc_pos(w_hbm, out_hbm, vbuf):
        wid = lax.axis_index("s") * nc + lax.axis_index("c")
        base = wid * rows

        def body(i, _):
            s0 = base + i * chunk
            pltpu.sync_copy(w_hbm.at[pl.ds(s0, chunk)], vbuf)
            for b in range(batch):
                pltpu.sync_copy(vbuf, out_hbm.at[pl.ds(b * seq + s0, chunk)])
            return ()

        lax.fori_loop(0, n_chunks, body, ())

    return sc_pos


def kernel(tokens, token_embed, W_pos):
    batch, seq, d = token_embed.shape
    pos_flat = _make_sc_pos(batch, seq, d, W_pos.dtype)(W_pos)
    pos_embed = pos_flat.reshape(batch, seq, d)

    block_s = 512
    te_out = pl.pallas_call(
        _copy_kernel,
        grid=(seq // block_s,),
        in_specs=[pl.BlockSpec((batch, block_s, d), lambda j: (0, j, 0))],
        out_specs=pl.BlockSpec((batch, block_s, d), lambda j: (0, j, 0)),
        out_shape=jax.ShapeDtypeStruct((batch, seq, d), token_embed.dtype),
    )(token_embed)
    return (pos_embed, te_out)
